# Initial kernel scaffold; baseline (speedup 1.0000x reference)
import jax
import jax.numpy as jnp
from jax.experimental import pallas as pl


def kernel(node_feature, edge_index, distance, is_same_team, Wd, bd, emb, W_ni1, W_fij1, W_nj1, b_e1, attn1, W_node1, W_ni2, W_fij2, W_nj2, b_e2, attn2, W_node2):
    raise NotImplementedError("write your pallas kernel here")


# speedup vs baseline: 4.3129x; 4.3129x over previous
import functools

import jax
import jax.numpy as jnp
from jax.experimental import pallas as pl
from jax.experimental.pallas import tpu as pltpu

_E_TILE = 1600


def _leaky(x):
    return jnp.where(x >= 0, x, 0.01 * x)


def _phase1_body(feat_ref, wcat_ref, bias_ref, wred_ref, out_ref):
    feat = feat_ref[...]
    f_tmp = jnp.dot(feat, wcat_ref[...], preferred_element_type=jnp.float32)
    f_tmp = f_tmp + bias_ref[...]
    f_out = _leaky(f_tmp)
    red = jnp.dot(f_out, wred_ref[...], preferred_element_type=jnp.float32)
    out_ref[...] = jnp.concatenate([jnp.exp(red[:, :4]), red[:, 4:]], axis=1)


def kernel(node_feature, edge_index, distance, is_same_team, Wd, bd, emb,
           W_ni1, W_fij1, W_nj1, b_e1, attn1, W_node1,
           W_ni2, W_fij2, W_nj2, b_e2, attn2, W_node2):
    E = edge_index.shape[1]
    N = node_feature.shape[0]
    src = edge_index[0]
    dst = edge_index[1]
    istf = is_same_team[:, 0].astype(jnp.float32)

    Wf1a = W_fij1[:32]
    Wf1b = W_fij1[32:]
    v1 = (Wd @ Wf1a)[0]
    T = emb @ Wf1b
    bias1 = bd @ Wf1a + b_e1 + T[0]
    dT = T[1] - T[0]
    Wcat = jnp.concatenate([W_ni1, W_nj1, v1[None], dT[None]], axis=0)
    A4 = attn1[0]
    Wred1 = (jnp.eye(4, dtype=jnp.float32)[:, None, :] * A4[:, :, None]).reshape(256, 4)
    Wtile = jnp.concatenate([W_fij2] * 4, axis=0) / 4.0
    Wred = jnp.concatenate([Wred1, Wtile], axis=1)
    WG = jnp.einsum('chd,dk->chk', W_node1.reshape(3, 4, 64), W_ni2) / 4.0
    WH = jnp.einsum('chd,dk->chk', W_node1.reshape(3, 4, 64), W_nj2) / 4.0
    G = jnp.einsum('nc,chk->nhk', node_feature, WG)
    H = jnp.einsum('nc,chk->nhk', node_feature, WH)

    hs = jnp.take(node_feature, src, axis=0)
    hd = jnp.take(node_feature, dst, axis=0)
    feat = jnp.concatenate([hs, hd, distance, istf[:, None]], axis=1)

    grid = E // _E_TILE
    O = pl.pallas_call(
        _phase1_body,
        grid=(grid,),
        in_specs=[
            pl.BlockSpec((_E_TILE, 8), lambda i: (i, 0)),
            pl.BlockSpec((8, 256), lambda i: (0, 0)),
            pl.BlockSpec((1, 256), lambda i: (0, 0)),
            pl.BlockSpec((256, 8), lambda i: (0, 0)),
        ],
        out_specs=pl.BlockSpec((_E_TILE, 8), lambda i: (i, 0)),
        out_shape=jax.ShapeDtypeStruct((E, 8), jnp.float32),
    )(feat, Wcat, bias1[None], Wred)
    ex = O[:, :4]
    Rm = O[:, 4:]

    denom = jax.ops.segment_sum(ex, dst, num_segments=N)
    a = ex / (jnp.take(denom, dst, axis=0) + 1e-16)
    pg = jnp.einsum('eh,ehk->ek', a, jnp.take(G, src, axis=0))
    qg = jnp.einsum('eh,ehk->ek', a, jnp.take(H, src, axis=0))
    P = jax.ops.segment_sum(pg, dst, num_segments=N)
    Q = jax.ops.segment_sum(qg, dst, num_segments=N)
    out = _leaky(jnp.take(P, src, axis=0) + jnp.take(Q, dst, axis=0) + Rm + b_e2)
    return out.mean(axis=-1, keepdims=True)


# speedup vs baseline: 25.9028x; 6.0058x over previous
import functools

import jax
import jax.numpy as jnp
from jax import lax
from jax.experimental import pallas as pl
from jax.experimental.pallas import tpu as pltpu
from jax.experimental.pallas import tpu_sc as plsc

N_NODES = 50000
E_REAL = 800000
NW = 32
EP = 819200
EW = EP // NW
CB = 1024
KB = CB // 128
NCHUNK = EW // CB
NACC = 50048
RT = NACC // 16
TE = 2048
NT = 2000

_MESH = plsc.VectorSubcoreMesh(core_axis_name="c", subcore_axis_name="s")
_SC_PARAMS = pltpu.CompilerParams(use_tc_tiling_on_sc=False,
                                  needs_layout_passes=False)


def _leaky(x):
    return jnp.where(x >= 0, x, 0.01 * x)


def _splat(c):
    return jnp.full((16,), c, jnp.int32)


def _t0_body(h8_ref, w4_ref, gh_ref):
    gh_ref[...] = jnp.dot(h8_ref[...], w4_ref[...],
                          preferred_element_type=jnp.float32)


def _t1_body(hs_ref, hd_ref, d_ref, t_ref, wni_ref, wnj_ref, vrow_ref,
             trow_ref, bias_ref, b2_ref, wred_ref, exrm_ref):
    i = pl.program_id(0)
    f_tmp = (jnp.dot(hs_ref[...], wni_ref[...],
                     preferred_element_type=jnp.float32)
             + jnp.dot(hd_ref[...], wnj_ref[...],
                       preferred_element_type=jnp.float32)
             + d_ref[...] * vrow_ref[...]
             + t_ref[...] * trow_ref[...]
             + bias_ref[...])
    f_out = _leaky(f_tmp)
    red = jnp.dot(f_out, wred_ref[...], preferred_element_type=jnp.float32)
    rows = i * TE + lax.broadcasted_iota(jnp.int32, (TE, 1), 0)
    ex = jnp.where(rows < E_REAL, jnp.exp(red[:, :4]), 0.0)
    exrm_ref[...] = jnp.concatenate([ex, red[:, 4:] + b2_ref[...]], axis=1)


def _t2_body(d0_ref, d1_ref, out_ref):
    out_ref[...] = 1.0 / (d0_ref[...] + d1_ref[...] + 1e-16)


def _t3_body(p0_ref, p1_ref, out_ref):
    out_ref[...] = p0_ref[...] + p1_ref[...]


def _worker_id():
    return lax.axis_index("c") * 16 + lax.axis_index("s")


def _s1_body(h8, srcg, dstg, hs_out, hd_out, idxs_v, idxd_v, hsb, hdb, sem):
    wid = _worker_id()

    def chunk(jc, carry):
        base = wid * EW + jc * CB
        rowb = wid * (EW // 128) + jc * KB
        pltpu.sync_copy(srcg.at[pl.ds(rowb, KB)], idxs_v)
        pltpu.sync_copy(dstg.at[pl.ds(rowb, KB)], idxd_v)
        cps = []
        for j in range(KB):
            cps.append(pltpu.async_copy(
                h8.at[idxs_v.at[j]], hsb.at[pl.ds(j * 128, 128)], sem))
            cps.append(pltpu.async_copy(
                h8.at[idxd_v.at[j]], hdb.at[pl.ds(j * 128, 128)], sem))
        for cp in cps:
            cp.wait()
        pltpu.sync_copy(hsb, hs_out.at[pl.ds(base, CB)])
        pltpu.sync_copy(hdb, hd_out.at[pl.ds(base, CB)])
        return carry

    lax.fori_loop(0, NCHUNK, chunk, 0)


def _s2_body(dstg, exrm, zeros8, dp_out, idxd_v, exb, shared):
    cid = lax.axis_index("c")
    sid = lax.axis_index("s")
    wid = cid * 16 + sid
    pltpu.sync_copy(zeros8.at[pl.ds(sid * RT, RT)],
                    shared.at[pl.ds(sid * RT, RT)])
    plsc.subcore_barrier()

    def chunk(jc, carry):
        base = wid * EW + jc * CB
        rowb = wid * (EW // 128) + jc * KB
        pltpu.sync_copy(dstg.at[pl.ds(rowb, KB)], idxd_v)
        pltpu.sync_copy(exrm.at[pl.ds(base, CB)], exb)
        for j in range(KB):
            pltpu.sync_copy(exb.at[pl.ds(j * 128, 128)],
                            shared.at[idxd_v.at[j]], add=True)
        return carry

    lax.fori_loop(0, NCHUNK, chunk, 0)
    plsc.subcore_barrier()
    pltpu.sync_copy(shared.at[pl.ds(sid * RT, RT)],
                    dp_out.at[cid, pl.ds(sid * RT, RT)])


def _s3_body(srcg, dstg, exrm, dr_hbm, gh_hbm, zeros8, pqp_out,
             idxs_v, idxd_v, exb, drows, ghrows, pqb, shared8, sem):
    cid = lax.axis_index("c")
    sid = lax.axis_index("s")
    wid = cid * 16 + sid
    pltpu.sync_copy(zeros8.at[pl.ds(sid * RT, RT)],
                    shared8.at[pl.ds(sid * RT, RT)])
    plsc.subcore_barrier()
    iota = lax.iota(jnp.int32, 16)

    def chunk(jc, carry):
        base = wid * EW + jc * CB
        rowb = wid * (EW // 128) + jc * KB
        pltpu.sync_copy(srcg.at[pl.ds(rowb, KB)], idxs_v)
        pltpu.sync_copy(dstg.at[pl.ds(rowb, KB)], idxd_v)
        pltpu.sync_copy(exrm.at[pl.ds(base, CB)], exb)
        cps = []
        for j in range(KB):
            cps.append(pltpu.async_copy(
                dr_hbm.at[idxd_v.at[j]], drows.at[pl.ds(j * 128, 128)], sem))
            cps.append(pltpu.async_copy(
                gh_hbm.at[idxs_v.at[j]], ghrows.at[pl.ds(j * 128, 128)], sem))
        for cp in cps:
            cp.wait()

        def grp(g, c2):
            e16 = g * 16 + iota
            a = [plsc.load_gather(exb, [e16, _splat(h)])
                 * plsc.load_gather(drows, [e16, _splat(h)])
                 for h in range(4)]
            for k in range(4):
                accg = a[0] * plsc.load_gather(ghrows, [e16, _splat(k)])
                acch = a[0] * plsc.load_gather(ghrows, [e16, _splat(16 + k)])
                for h in range(1, 4):
                    accg += a[h] * plsc.load_gather(
                        ghrows, [e16, _splat(h * 4 + k)])
                    acch += a[h] * plsc.load_gather(
                        ghrows, [e16, _splat(16 + h * 4 + k)])
                plsc.store_scatter(pqb, [e16, _splat(k)], accg)
                plsc.store_scatter(pqb, [e16, _splat(4 + k)], acch)
            return c2

        lax.fori_loop(0, CB // 16, grp, 0)
        for j in range(KB):
            pltpu.sync_copy(pqb.at[pl.ds(j * 128, 128)],
                            shared8.at[idxd_v.at[j]], add=True)
        return carry

    lax.fori_loop(0, NCHUNK, chunk, 0)
    plsc.subcore_barrier()
    pltpu.sync_copy(shared8.at[pl.ds(sid * RT, RT)],
                    pqp_out.at[cid, pl.ds(sid * RT, RT)])


def _s4_body(srcg, dstg, exrm, pq_hbm, out_hbm,
             idxs_v, idxd_v, exb, pqs, pqd, outb, sem):
    wid = _worker_id()
    iota = lax.iota(jnp.int32, 16)

    def chunk(jc, carry):
        base = wid * EW + jc * CB
        rowb = wid * (EW // 128) + jc * KB
        pltpu.sync_copy(srcg.at[pl.ds(rowb, KB)], idxs_v)
        pltpu.sync_copy(dstg.at[pl.ds(rowb, KB)], idxd_v)
        pltpu.sync_copy(exrm.at[pl.ds(base, CB)], exb)
        cps = []
        for j in range(KB):
            cps.append(pltpu.async_copy(
                pq_hbm.at[idxs_v.at[j]], pqs.at[pl.ds(j * 128, 128)], sem))
            cps.append(pltpu.async_copy(
                pq_hbm.at[idxd_v.at[j]], pqd.at[pl.ds(j * 128, 128)], sem))
        for cp in cps:
            cp.wait()

        def grp(g, c2):
            e16 = g * 16 + iota
            acc = jnp.zeros((16,), jnp.float32)
            for k in range(4):
                x = (plsc.load_gather(pqs, [e16, _splat(k)])
                     + plsc.load_gather(pqd, [e16, _splat(4 + k)])
                     + plsc.load_gather(exb, [e16, _splat(4 + k)]))
                acc += jnp.maximum(x, 0.0) + 0.01 * jnp.minimum(x, 0.0)
            plsc.store_scatter(outb, [e16], acc)
            return c2

        lax.fori_loop(0, CB // 16, grp, 0)
        pltpu.sync_copy(outb, out_hbm.at[pl.ds(base, CB)])
        return carry

    lax.fori_loop(0, NCHUNK, chunk, 0)


def kernel(node_feature, edge_index, distance, is_same_team, Wd, bd, emb,
           W_ni1, W_fij1, W_nj1, b_e1, attn1, W_node1,
           W_ni2, W_fij2, W_nj2, b_e2, attn2, W_node2):
    E = edge_index.shape[1]
    N = node_feature.shape[0]
    f32 = jnp.float32

    Wf1a = W_fij1[:32]
    Wf1b = W_fij1[32:]
    v1 = (Wd @ Wf1a)[0]
    T = emb @ Wf1b
    bias1 = bd @ Wf1a + b_e1 + T[0]
    dT = T[1] - T[0]
    zrow = jnp.zeros((5, 256), f32)
    Wni_p = jnp.concatenate([W_ni1, zrow], axis=0)
    Wnj_p = jnp.concatenate([W_nj1, zrow], axis=0)
    A4 = attn1[0]
    Wred1 = (jnp.eye(4, dtype=f32)[:, None, :] * A4[:, :, None]).reshape(256, 4)
    Wtile = jnp.concatenate([W_fij2] * 4, axis=0) / 16.0
    Wred = jnp.concatenate([Wred1, Wtile], axis=1)
    b2q = (b_e2 / 4.0)[None]
    WG = jnp.einsum('chd,dk->chk', W_node1.reshape(3, 4, 64), W_ni2) / 16.0
    WH = jnp.einsum('chd,dk->chk', W_node1.reshape(3, 4, 64), W_nj2) / 16.0
    W4 = jnp.concatenate([WG.reshape(3, 16), WH.reshape(3, 16)],
                         axis=1)
    W4 = jnp.concatenate([W4, jnp.zeros((5, 32), f32)], axis=0)

    h8 = jnp.pad(node_feature, ((0, 0), (0, 5)))
    pe = EP - E
    srcg = jnp.pad(edge_index[0], (0, pe)).reshape(EP // 128, 128)
    dstg = jnp.pad(edge_index[1], (0, pe)).reshape(EP // 128, 128)
    distp = jnp.pad(distance, ((0, pe), (0, 0)))
    istp = jnp.pad(is_same_team.astype(f32), ((0, pe), (0, 0)))
    zeros8 = jnp.zeros((NACC, 8), f32)

    gh = pl.pallas_call(
        _t0_body,
        grid=(N // NT,),
        in_specs=[pl.BlockSpec((NT, 8), lambda i: (i, 0)),
                  pl.BlockSpec((8, 32), lambda i: (0, 0))],
        out_specs=pl.BlockSpec((NT, 32), lambda i: (i, 0)),
        out_shape=jax.ShapeDtypeStruct((N, 32), f32),
    )(h8, W4)

    s1 = pl.kernel(
        _s1_body,
        out_type=(jax.ShapeDtypeStruct((EP, 8), f32),
                  jax.ShapeDtypeStruct((EP, 8), f32)),
        mesh=_MESH,
        compiler_params=_SC_PARAMS,
        scratch_types=[
            pltpu.VMEM((KB, 128), jnp.int32),
            pltpu.VMEM((KB, 128), jnp.int32),
            pltpu.VMEM((CB, 8), f32),
            pltpu.VMEM((CB, 8), f32),
            pltpu.SemaphoreType.DMA,
        ],
    )
    hs, hd = s1(h8, srcg, dstg)

    exrm = pl.pallas_call(
        _t1_body,
        grid=(EP // TE,),
        in_specs=[
            pl.BlockSpec((TE, 8), lambda i: (i, 0)),
            pl.BlockSpec((TE, 8), lambda i: (i, 0)),
            pl.BlockSpec((TE, 1), lambda i: (i, 0)),
            pl.BlockSpec((TE, 1), lambda i: (i, 0)),
            pl.BlockSpec((8, 256), lambda i: (0, 0)),
            pl.BlockSpec((8, 256), lambda i: (0, 0)),
            pl.BlockSpec((1, 256), lambda i: (0, 0)),
            pl.BlockSpec((1, 256), lambda i: (0, 0)),
            pl.BlockSpec((1, 256), lambda i: (0, 0)),
            pl.BlockSpec((1, 4), lambda i: (0, 0)),
            pl.BlockSpec((256, 8), lambda i: (0, 0)),
        ],
        out_specs=pl.BlockSpec((TE, 8), lambda i: (i, 0)),
        out_shape=jax.ShapeDtypeStruct((EP, 8), f32),
    )(hs, hd, distp, istp, Wni_p, Wnj_p, v1[None], dT[None], bias1[None],
      b2q, Wred)

    s2 = pl.kernel(
        _s2_body,
        out_type=jax.ShapeDtypeStruct((2, NACC, 8), f32),
        mesh=_MESH,
        compiler_params=_SC_PARAMS,
        scratch_types=[
            pltpu.VMEM((KB, 128), jnp.int32),
            pltpu.VMEM((CB, 8), f32),
            pltpu.VMEM_SHARED((NACC, 8), f32),
        ],
    )
    dp = s2(dstg, exrm, zeros8)

    r8 = NACC * 8 // 512
    dr = pl.pallas_call(
        _t2_body,
        in_specs=[pl.BlockSpec((r8, 512), lambda: (0, 0)),
                  pl.BlockSpec((r8, 512), lambda: (0, 0))],
        out_specs=pl.BlockSpec((r8, 512), lambda: (0, 0)),
        out_shape=jax.ShapeDtypeStruct((r8, 512), f32),
    )(dp[0].reshape(r8, 512), dp[1].reshape(r8, 512)).reshape(NACC, 8)

    s3 = pl.kernel(
        _s3_body,
        out_type=jax.ShapeDtypeStruct((2, NACC, 8), f32),
        mesh=_MESH,
        compiler_params=_SC_PARAMS,
        scratch_types=[
            pltpu.VMEM((KB, 128), jnp.int32),
            pltpu.VMEM((KB, 128), jnp.int32),
            pltpu.VMEM((CB, 8), f32),
            pltpu.VMEM((CB, 8), f32),
            pltpu.VMEM((CB, 32), f32),
            pltpu.VMEM((CB, 8), f32),
            pltpu.VMEM_SHARED((NACC, 8), f32),
            pltpu.SemaphoreType.DMA,
        ],
    )
    pqp = s3(srcg, dstg, exrm, dr, gh, zeros8)

    pq = pl.pallas_call(
        _t3_body,
        in_specs=[pl.BlockSpec((r8, 512), lambda: (0, 0)),
                  pl.BlockSpec((r8, 512), lambda: (0, 0))],
        out_specs=pl.BlockSpec((r8, 512), lambda: (0, 0)),
        out_shape=jax.ShapeDtypeStruct((r8, 512), f32),
    )(pqp[0].reshape(r8, 512), pqp[1].reshape(r8, 512)).reshape(NACC, 8)

    s4 = pl.kernel(
        _s4_body,
        out_type=jax.ShapeDtypeStruct((EP,), f32),
        mesh=_MESH,
        compiler_params=_SC_PARAMS,
        scratch_types=[
            pltpu.VMEM((KB, 128), jnp.int32),
            pltpu.VMEM((KB, 128), jnp.int32),
            pltpu.VMEM((CB, 8), f32),
            pltpu.VMEM((CB, 8), f32),
            pltpu.VMEM((CB, 8), f32),
            pltpu.VMEM((CB,), f32),
            pltpu.SemaphoreType.DMA,
        ],
    )
    out = s4(srcg, dstg, exrm, pq)
    return out[:E, None]
